# baseline (speedup 1.0000x reference)
import functools

import jax
import jax.numpy as jnp
from jax import lax
from jax.experimental import pallas as pl
from jax.experimental.pallas import tpu as pltpu
from jax.experimental.pallas import tpu_sc as plsc

B = 16384
D = 64
N = 1000000
W = 4096
NG = (N + 2 * W - 1) // (2 * W)
NP = NG * W
DP = 2 * D
NC = 2
NS = 16
NW_ = NC * NS
BPW = B // NW_
CHUNK = 128
NCHUNK = BPW // CHUNK


def _tc_transpose_body(x_ref, o_ref):
    x = x_ref[...]
    o_ref[...] = jnp.concatenate([x[:, :W].T, x[:, W:].T], axis=1)


def _tc_transpose(wt):
    return pl.pallas_call(
        _tc_transpose_body,
        grid=(NG,),
        in_specs=[pl.BlockSpec((D, 2 * W), lambda i: (0, i))],
        out_specs=pl.BlockSpec((W, DP), lambda i: (i, 0)),
        out_shape=jax.ShapeDtypeStruct((NP, DP), jnp.float32),
    )(wt)


def _sc_gather(uid3, iid3, u_pairs, item_tbl):
    mesh = plsc.VectorSubcoreMesh(
        core_axis_name="c", subcore_axis_name="s", num_cores=NC, num_subcores=NS
    )

    @functools.partial(
        pl.kernel,
        out_type=[
            jax.ShapeDtypeStruct((B, DP), jnp.float32),
            jax.ShapeDtypeStruct((B, D), jnp.float32),
        ],
        mesh=mesh,
        scratch_types=[
            pltpu.VMEM((NCHUNK, CHUNK), jnp.int32),
            pltpu.VMEM((NCHUNK, CHUNK), jnp.int32),
            pltpu.VMEM((2, CHUNK, DP), jnp.float32),
            pltpu.VMEM((2, CHUNK, D), jnp.float32),
            pltpu.SemaphoreType.DMA((2, 2)),
            pltpu.SemaphoreType.DMA((2, 2)),
        ],
        compiler_params=pltpu.CompilerParams(use_tc_tiling_on_sc=False),
    )
    def k(u_tbl, i_tbl, uid_hbm, iid_hbm, u_out, i_out, uid_v, iid_v, u_rows, i_rows, gsem, osem):
        wid = lax.axis_index("s") * NC + lax.axis_index("c")
        base = wid * BPW
        pltpu.sync_copy(uid_hbm.at[wid], uid_v)
        pltpu.sync_copy(iid_hbm.at[wid], iid_v)
        gathers = [None, None]
        drains = [None, None]
        for j in range(NCHUNK):
            s = j % 2
            if drains[s] is not None:
                for c in drains[s]:
                    c.wait()
                drains[s] = None
            gathers[s] = (
                pltpu.async_copy(u_tbl.at[uid_v.at[j]], u_rows.at[s], gsem.at[s, 0]),
                pltpu.async_copy(i_tbl.at[iid_v.at[j]], i_rows.at[s], gsem.at[s, 1]),
            )
            if j >= 1:
                sp = (j - 1) % 2
                for c in gathers[sp]:
                    c.wait()
                gathers[sp] = None
                dst = pl.ds(base + (j - 1) * CHUNK, CHUNK)
                drains[sp] = (
                    pltpu.async_copy(u_rows.at[sp], u_out.at[dst], osem.at[sp, 0]),
                    pltpu.async_copy(i_rows.at[sp], i_out.at[dst], osem.at[sp, 1]),
                )
        s = (NCHUNK - 1) % 2
        for c in gathers[s]:
            c.wait()
        dst = pl.ds(base + (NCHUNK - 1) * CHUNK, CHUNK)
        drains[s] = (
            pltpu.async_copy(u_rows.at[s], u_out.at[dst], osem.at[s, 0]),
            pltpu.async_copy(i_rows.at[s], i_out.at[dst], osem.at[s, 1]),
        )
        for d in drains:
            if d is not None:
                for c in d:
                    c.wait()

    return k(u_pairs, item_tbl, uid3, iid3)


def _tc_dot_body(u_ref, i_ref, up_ref, o_ref):
    u2 = u_ref[...]
    i1 = i_ref[...]
    shp = o_ref.shape
    lo = jnp.sum(u2[:, :D] * i1, axis=1).reshape(shp)
    hi = jnp.sum(u2[:, D:] * i1, axis=1).reshape(shp)
    up = up_ref[...]
    o_ref[...] = (1.0 - up) * lo + up * hi


def _tc_dot(u_e, i_e, u_par):
    rows_per_blk = 2048
    grid = (B // rows_per_blk,)
    out = pl.pallas_call(
        _tc_dot_body,
        grid=grid,
        in_specs=[
            pl.BlockSpec((rows_per_blk, DP), lambda i: (i, 0)),
            pl.BlockSpec((rows_per_blk, D), lambda i: (i, 0)),
            pl.BlockSpec((rows_per_blk // 128, 128), lambda i: (i, 0)),
        ],
        out_specs=pl.BlockSpec((rows_per_blk // 128, 128), lambda i: (i, 0)),
        out_shape=jax.ShapeDtypeStruct((B // 128, 128), jnp.float32),
    )(u_e, i_e, u_par)
    return out.reshape(B)


def kernel(u_ids, i_ids, user_weight, item_weight):
    u_pairs = _tc_transpose(user_weight.T)
    u_pair_idx = (u_ids >> 13) * W + (u_ids & (W - 1))
    uid3 = u_pair_idx.reshape(NW_, NCHUNK, CHUNK)
    iid3 = i_ids.reshape(NW_, NCHUNK, CHUNK)
    u_e, i_e = _sc_gather(uid3, iid3, u_pairs, item_weight)
    u_par = ((u_ids >> 12) & 1).astype(jnp.float32).reshape(B // 128, 128)
    return _tc_dot(u_e, i_e, u_par)


# speedup vs baseline: 1.7688x; 1.7688x over previous
import functools

import jax
import jax.numpy as jnp
from jax import lax
from jax.experimental import pallas as pl
from jax.experimental.pallas import tpu as pltpu
from jax.experimental.pallas import tpu_sc as plsc

B = 16384
D = 64
N = 1000000
W = 4096
NG = (N + 2 * W - 1) // (2 * W)
NP = NG * W
DP = 2 * D
NC = 2
NS = 16
NW_ = NC * NS
BPW = B // NW_
CHUNK = 128
NCHUNK = BPW // CHUNK


def _tc_transpose_body(x_ref, o_ref):
    x = x_ref[...]
    o_ref[...] = jnp.concatenate([x[:, :W], x[:, W:]], axis=0).T


def _tc_transpose(wt):
    return pl.pallas_call(
        _tc_transpose_body,
        grid=(NG,),
        in_specs=[pl.BlockSpec((D, 2 * W), lambda i: (0, i))],
        out_specs=pl.BlockSpec((W, DP), lambda i: (i, 0)),
        out_shape=jax.ShapeDtypeStruct((NP, DP), jnp.float32),
    )(wt)


def _sc_gather(uid3, iid3, u_pairs, i_pairs):
    mesh = plsc.VectorSubcoreMesh(
        core_axis_name="c", subcore_axis_name="s", num_cores=NC, num_subcores=NS
    )
    row_t = jax.ShapeDtypeStruct((B, DP), jnp.float32)

    @functools.partial(
        pl.kernel,
        out_type=[row_t, row_t],
        mesh=mesh,
        scratch_types=[
            pltpu.VMEM((NCHUNK, CHUNK), jnp.int32),
            pltpu.VMEM((NCHUNK, CHUNK), jnp.int32),
            pltpu.VMEM((2, CHUNK, DP), jnp.float32),
            pltpu.VMEM((2, CHUNK, DP), jnp.float32),
            pltpu.SemaphoreType.DMA((2, 2)),
            pltpu.SemaphoreType.DMA((2, 2)),
        ],
    )
    def k(u_tbl, i_tbl, uid_hbm, iid_hbm, u_out, i_out, uid_v, iid_v, u_rows, i_rows, gsem, osem):
        wid = lax.axis_index("s") * NC + lax.axis_index("c")
        base = wid * BPW
        pltpu.sync_copy(uid_hbm.at[wid], uid_v)
        pltpu.sync_copy(iid_hbm.at[wid], iid_v)
        gathers = [None, None]
        drains = [None, None]
        for j in range(NCHUNK):
            s = j % 2
            if drains[s] is not None:
                for c in drains[s]:
                    c.wait()
                drains[s] = None
            gathers[s] = (
                pltpu.async_copy(u_tbl.at[uid_v.at[j]], u_rows.at[s], gsem.at[s, 0]),
                pltpu.async_copy(i_tbl.at[iid_v.at[j]], i_rows.at[s], gsem.at[s, 1]),
            )
            if j >= 1:
                sp = (j - 1) % 2
                for c in gathers[sp]:
                    c.wait()
                gathers[sp] = None
                dst = pl.ds(base + (j - 1) * CHUNK, CHUNK)
                drains[sp] = (
                    pltpu.async_copy(u_rows.at[sp], u_out.at[dst], osem.at[sp, 0]),
                    pltpu.async_copy(i_rows.at[sp], i_out.at[dst], osem.at[sp, 1]),
                )
        s = (NCHUNK - 1) % 2
        for c in gathers[s]:
            c.wait()
        dst = pl.ds(base + (NCHUNK - 1) * CHUNK, CHUNK)
        drains[s] = (
            pltpu.async_copy(u_rows.at[s], u_out.at[dst], osem.at[s, 0]),
            pltpu.async_copy(i_rows.at[s], i_out.at[dst], osem.at[s, 1]),
        )
        for d in drains:
            if d is not None:
                for c in d:
                    c.wait()

    return k(u_pairs, i_pairs, uid3, iid3)


def _tc_dot_body(u_ref, i_ref, up_ref, ip_ref, o_ref):
    u2 = u_ref[...]
    i2 = i_ref[...]
    ul, uh = u2[:, :D], u2[:, D:]
    il, ih = i2[:, :D], i2[:, D:]
    shp = o_ref.shape
    ll = jnp.sum(ul * il, axis=1).reshape(shp)
    lh = jnp.sum(ul * ih, axis=1).reshape(shp)
    hl = jnp.sum(uh * il, axis=1).reshape(shp)
    hh = jnp.sum(uh * ih, axis=1).reshape(shp)
    up = up_ref[...]
    ip = ip_ref[...]
    o_ref[...] = (
        (1.0 - up) * ((1.0 - ip) * ll + ip * lh)
        + up * ((1.0 - ip) * hl + ip * hh)
    )


def _tc_dot(u_e, i_e, u_par, i_par):
    rows_per_blk = 2048
    grid = (B // rows_per_blk,)
    out = pl.pallas_call(
        _tc_dot_body,
        grid=grid,
        in_specs=[
            pl.BlockSpec((rows_per_blk, DP), lambda i: (i, 0)),
            pl.BlockSpec((rows_per_blk, DP), lambda i: (i, 0)),
            pl.BlockSpec((rows_per_blk // 128, 128), lambda i: (i, 0)),
            pl.BlockSpec((rows_per_blk // 128, 128), lambda i: (i, 0)),
        ],
        out_specs=pl.BlockSpec((rows_per_blk // 128, 128), lambda i: (i, 0)),
        out_shape=jax.ShapeDtypeStruct((B // 128, 128), jnp.float32),
    )(u_e, i_e, u_par, i_par)
    return out.reshape(B)


def kernel(u_ids, i_ids, user_weight, item_weight):
    u_pairs = _tc_transpose(user_weight.T)
    i_pairs = _tc_transpose(item_weight.T)
    u_pair_idx = (u_ids >> 13) * W + (u_ids & (W - 1))
    i_pair_idx = (i_ids >> 13) * W + (i_ids & (W - 1))
    uid3 = u_pair_idx.reshape(NW_, NCHUNK, CHUNK)
    iid3 = i_pair_idx.reshape(NW_, NCHUNK, CHUNK)
    u_e, i_e = _sc_gather(uid3, iid3, u_pairs, i_pairs)
    u_par = ((u_ids >> 12) & 1).astype(jnp.float32).reshape(B // 128, 128)
    i_par = ((i_ids >> 12) & 1).astype(jnp.float32).reshape(B // 128, 128)
    return _tc_dot(u_e, i_e, u_par, i_par)


# speedup vs baseline: 2.0326x; 1.1491x over previous
import functools

import jax
import jax.numpy as jnp
from jax import lax
from jax.experimental import pallas as pl
from jax.experimental.pallas import tpu as pltpu
from jax.experimental.pallas import tpu_sc as plsc

B = 16384
D = 64
N = 1000000
W = 8192
NG = (N + 2 * W - 1) // (2 * W)
NP = NG * W
DP = 2 * D
NC = 2
NS = 16
NW_ = NC * NS
BPW = B // NW_
CHUNK = 128
NCHUNK = BPW // CHUNK


def _tc_transpose_body(x_ref, o_ref):
    x = x_ref[...]
    o_ref[...] = jnp.concatenate([x[:, :W], x[:, W:]], axis=0).T


def _tc_transpose(wt):
    return pl.pallas_call(
        _tc_transpose_body,
        grid=(NG,),
        in_specs=[pl.BlockSpec((D, 2 * W), lambda i: (0, i))],
        out_specs=pl.BlockSpec((W, DP), lambda i: (i, 0)),
        out_shape=jax.ShapeDtypeStruct((NP, DP), jnp.float32),
    )(wt)


def _sc_gather(uid3, iid3, u_pairs, i_pairs):
    mesh = plsc.VectorSubcoreMesh(
        core_axis_name="c", subcore_axis_name="s", num_cores=NC, num_subcores=NS
    )
    row_t = jax.ShapeDtypeStruct((B, DP), jnp.float32)

    @functools.partial(
        pl.kernel,
        out_type=[row_t, row_t],
        mesh=mesh,
        scratch_types=[
            pltpu.VMEM((NCHUNK, CHUNK), jnp.int32),
            pltpu.VMEM((NCHUNK, CHUNK), jnp.int32),
            pltpu.VMEM((2, CHUNK, DP), jnp.float32),
            pltpu.VMEM((2, CHUNK, DP), jnp.float32),
            pltpu.SemaphoreType.DMA((2, 2)),
            pltpu.SemaphoreType.DMA((2, 2)),
        ],
    )
    def k(u_tbl, i_tbl, uid_hbm, iid_hbm, u_out, i_out, uid_v, iid_v, u_rows, i_rows, gsem, osem):
        wid = lax.axis_index("s") * NC + lax.axis_index("c")
        base = wid * BPW
        pltpu.sync_copy(uid_hbm.at[wid], uid_v)
        pltpu.sync_copy(iid_hbm.at[wid], iid_v)
        gathers = [None, None]
        drains = [None, None]
        for j in range(NCHUNK):
            s = j % 2
            if drains[s] is not None:
                for c in drains[s]:
                    c.wait()
                drains[s] = None
            gathers[s] = (
                pltpu.async_copy(u_tbl.at[uid_v.at[j]], u_rows.at[s], gsem.at[s, 0]),
                pltpu.async_copy(i_tbl.at[iid_v.at[j]], i_rows.at[s], gsem.at[s, 1]),
            )
            if j >= 1:
                sp = (j - 1) % 2
                for c in gathers[sp]:
                    c.wait()
                gathers[sp] = None
                dst = pl.ds(base + (j - 1) * CHUNK, CHUNK)
                drains[sp] = (
                    pltpu.async_copy(u_rows.at[sp], u_out.at[dst], osem.at[sp, 0]),
                    pltpu.async_copy(i_rows.at[sp], i_out.at[dst], osem.at[sp, 1]),
                )
        s = (NCHUNK - 1) % 2
        for c in gathers[s]:
            c.wait()
        dst = pl.ds(base + (NCHUNK - 1) * CHUNK, CHUNK)
        drains[s] = (
            pltpu.async_copy(u_rows.at[s], u_out.at[dst], osem.at[s, 0]),
            pltpu.async_copy(i_rows.at[s], i_out.at[dst], osem.at[s, 1]),
        )
        for d in drains:
            if d is not None:
                for c in d:
                    c.wait()

    return k(u_pairs, i_pairs, uid3, iid3)


def _tc_dot_body(u_ref, i_ref, up_ref, ip_ref, o_ref):
    u2 = u_ref[...]
    i2 = i_ref[...]
    ul, uh = u2[:, :D], u2[:, D:]
    il, ih = i2[:, :D], i2[:, D:]
    shp = o_ref.shape
    ll = jnp.sum(ul * il, axis=1).reshape(shp)
    lh = jnp.sum(ul * ih, axis=1).reshape(shp)
    hl = jnp.sum(uh * il, axis=1).reshape(shp)
    hh = jnp.sum(uh * ih, axis=1).reshape(shp)
    up = up_ref[...]
    ip = ip_ref[...]
    o_ref[...] = (
        (1.0 - up) * ((1.0 - ip) * ll + ip * lh)
        + up * ((1.0 - ip) * hl + ip * hh)
    )


def _tc_dot(u_e, i_e, u_par, i_par):
    rows_per_blk = 2048
    grid = (B // rows_per_blk,)
    out = pl.pallas_call(
        _tc_dot_body,
        grid=grid,
        in_specs=[
            pl.BlockSpec((rows_per_blk, DP), lambda i: (i, 0)),
            pl.BlockSpec((rows_per_blk, DP), lambda i: (i, 0)),
            pl.BlockSpec((rows_per_blk // 128, 128), lambda i: (i, 0)),
            pl.BlockSpec((rows_per_blk // 128, 128), lambda i: (i, 0)),
        ],
        out_specs=pl.BlockSpec((rows_per_blk // 128, 128), lambda i: (i, 0)),
        out_shape=jax.ShapeDtypeStruct((B // 128, 128), jnp.float32),
    )(u_e, i_e, u_par, i_par)
    return out.reshape(B)


def kernel(u_ids, i_ids, user_weight, item_weight):
    u_pairs = _tc_transpose(user_weight.T)
    i_pairs = _tc_transpose(item_weight.T)
    u_pair_idx = (u_ids >> 14) * W + (u_ids & (W - 1))
    i_pair_idx = (i_ids >> 14) * W + (i_ids & (W - 1))
    uid3 = u_pair_idx.reshape(NW_, NCHUNK, CHUNK)
    iid3 = i_pair_idx.reshape(NW_, NCHUNK, CHUNK)
    u_e, i_e = _sc_gather(uid3, iid3, u_pairs, i_pairs)
    u_par = ((u_ids >> 13) & 1).astype(jnp.float32).reshape(B // 128, 128)
    i_par = ((i_ids >> 13) & 1).astype(jnp.float32).reshape(B // 128, 128)
    return _tc_dot(u_e, i_e, u_par, i_par)
